# Initial kernel scaffold; baseline (speedup 1.0000x reference)
import jax
import jax.numpy as jnp
from jax.experimental import pallas as pl


def kernel(title_ids, token_ids, title_table, token_table):
    raise NotImplementedError("write your pallas kernel here")


# speedup vs baseline: 5.4325x; 5.4325x over previous
import functools

import jax
import jax.numpy as jnp
from jax.experimental import pallas as pl
from jax.experimental.pallas import tpu as pltpu
from jax.experimental.pallas import tpu_sc as plsc

B = 16384
V = 100001
T = 10000
D = 32
L = 20

GW = 128


def _sc_gather(title_table, title_ids_2d, token_table_z, token_ids_2d):
    mesh = plsc.VectorSubcoreMesh(core_axis_name="c", subcore_axis_name="s")
    out_type = (
        jax.ShapeDtypeStruct((B, D), jnp.float32),
        jax.ShapeDtypeStruct((B * L, D), jnp.float32),
    )

    @functools.partial(
        pl.kernel, out_type=out_type, mesh=mesh,
        compiler_params=pltpu.CompilerParams(use_tc_tiling_on_sc=False))
    def k(title_tab_hbm, title_idx_hbm, tok_tab_hbm, tok_idx_hbm,
          out_title_hbm, out_tok_hbm):
        def title_body(i_vmem, o_vmem):
            pltpu.sync_copy(title_tab_hbm.at[i_vmem.at[0]], o_vmem)

        pltpu.emit_pipeline(
            title_body,
            grid=(B // GW,),
            in_specs=[pl.BlockSpec((1, GW), lambda i: (0, i))],
            out_specs=[pl.BlockSpec((GW, D), lambda i: (i, 0))],
            core_axis_name=("c", "s"),
            dimension_semantics=(pltpu.PARALLEL,),
        )(title_idx_hbm, out_title_hbm)

        def tok_body(i_vmem, o_vmem):
            pltpu.sync_copy(tok_tab_hbm.at[i_vmem.at[0]], o_vmem)

        pltpu.emit_pipeline(
            tok_body,
            grid=(B * L // GW,),
            in_specs=[pl.BlockSpec((1, GW), lambda i: (0, i))],
            out_specs=[pl.BlockSpec((GW, D), lambda i: (i, 0))],
            core_axis_name=("c", "s"),
            dimension_semantics=(pltpu.PARALLEL,),
        )(tok_idx_hbm, out_tok_hbm)

    return k(title_table, title_ids_2d, token_table_z, token_ids_2d)


def _tc_combine(token_ids, title_emb, tok_rows):
    K = 1024

    def body(ids_ref, title_ref, tok_ref, out_ref):
        ids = ids_ref[...]
        cnt = jnp.sum((ids != 0).astype(jnp.float32), axis=1, keepdims=True)
        denom = jnp.maximum(cnt, 1.0)
        summed = jnp.sum(tok_ref[...].reshape(K, L, D), axis=1)
        out_ref[:, :D] = title_ref[...]
        out_ref[:, D:] = summed / denom

    return pl.pallas_call(
        body,
        grid=(B // K,),
        in_specs=[
            pl.BlockSpec((K, L), lambda i: (i, 0)),
            pl.BlockSpec((K, D), lambda i: (i, 0)),
            pl.BlockSpec((K * L, D), lambda i: (i, 0)),
        ],
        out_specs=pl.BlockSpec((K, 2 * D), lambda i: (i, 0)),
        out_shape=jax.ShapeDtypeStruct((B, 2 * D), jnp.float32),
    )(token_ids, title_emb, tok_rows)


def kernel(title_ids, token_ids, title_table, token_table):
    token_table_z = token_table.at[0].set(0.0)
    title_emb, tok_rows = _sc_gather(
        title_table,
        title_ids.astype(jnp.int32).reshape(1, B),
        token_table_z,
        token_ids.astype(jnp.int32).reshape(1, B * L),
    )
    return _tc_combine(token_ids, title_emb, tok_rows)


# speedup vs baseline: 7.5737x; 1.3941x over previous
import functools

import jax
import jax.numpy as jnp
from jax import lax
from jax.experimental import pallas as pl
from jax.experimental.pallas import tpu as pltpu
from jax.experimental.pallas import tpu_sc as plsc

B = 16384
V = 100001
T = 10000
D = 32
L = 20

NC = 2
NS = 16
NW = NC * NS
IPW = B // NW
GW = 128
TOK_W = IPW * L // GW
TIT_W = IPW // GW


def _sc_gather_pool(title_table, title_ids_1d, token_table_z, token_ids_1d,
                    row2item):
    mesh = plsc.VectorSubcoreMesh(core_axis_name="c", subcore_axis_name="s")
    out_type = (
        jax.ShapeDtypeStruct((B, D), jnp.float32),
        jax.ShapeDtypeStruct((B, D), jnp.float32),
    )

    @functools.partial(
        pl.kernel, out_type=out_type, mesh=mesh,
        scratch_types=[
            pltpu.VMEM((TOK_W, GW), jnp.int32),
            pltpu.VMEM((GW,), jnp.int32),
            pltpu.VMEM((GW, D), jnp.float32),
            pltpu.VMEM_SHARED((NS * IPW, D), jnp.float32),
            pltpu.SemaphoreType.DMA,
        ],
        compiler_params=pltpu.CompilerParams(use_tc_tiling_on_sc=False))
    def k(title_tab_hbm, title_idx_hbm, tok_tab_hbm, tok_idx_hbm, pat_hbm,
          out_title_hbm, out_pool_hbm, pat_v, idx_v, rows_v, acc_sh, sem):
        sid = lax.axis_index("s")
        wid = sid * NC + lax.axis_index("c")

        pltpu.sync_copy(pat_hbm, pat_v)

        off = jnp.broadcast_to(sid * IPW, (16,)).astype(jnp.int32)

        def obody(i, carry):
            a = i // (GW // 16)
            b = i % (GW // 16)
            pat_v[a, pl.ds(b * 16, 16)] = pat_v[a, pl.ds(b * 16, 16)] + off
            return carry

        lax.fori_loop(0, TOK_W * (GW // 16), obody, 0)

        zero = jnp.zeros((16,), jnp.float32)

        def zbody(i, carry):
            rows_v[i, pl.ds(0, 16)] = zero
            rows_v[i, pl.ds(16, 16)] = zero
            return carry

        lax.fori_loop(0, GW, zbody, 0)

        def zcopy(c, carry):
            pltpu.sync_copy(rows_v, acc_sh.at[pl.ds(sid * IPW + c * GW, GW)])
            return carry

        lax.fori_loop(0, IPW // GW, zcopy, 0)

        row0 = wid * IPW * L

        def wbody(w, carry):
            pltpu.sync_copy(tok_idx_hbm.at[pl.ds(row0 + w * GW, GW)], idx_v)
            pltpu.async_copy(tok_tab_hbm.at[idx_v], rows_v, sem).wait()
            pltpu.sync_copy(rows_v, acc_sh.at[pat_v.at[w]], add=True)
            return carry

        lax.fori_loop(0, TOK_W, wbody, 0)

        pltpu.sync_copy(acc_sh.at[pl.ds(sid * IPW, IPW)],
                        out_pool_hbm.at[pl.ds(wid * IPW, IPW)])

        def tbody(w, carry):
            base = wid * IPW + w * GW
            pltpu.sync_copy(title_idx_hbm.at[pl.ds(base, GW)], idx_v)
            pltpu.async_copy(title_tab_hbm.at[idx_v], rows_v, sem).wait()
            pltpu.sync_copy(rows_v, out_title_hbm.at[pl.ds(base, GW)])
            return carry

        lax.fori_loop(0, TIT_W, tbody, 0)

    return k(title_table, title_ids_1d, token_table_z, token_ids_1d, row2item)


def _tc_combine(token_ids, title_emb, pooled):
    K = 1024

    def body(ids_ref, title_ref, pool_ref, out_ref):
        ids = ids_ref[...]
        cnt = jnp.sum((ids != 0).astype(jnp.float32), axis=1, keepdims=True)
        denom = jnp.maximum(cnt, 1.0)
        out_ref[:, :D] = title_ref[...]
        out_ref[:, D:] = pool_ref[...] / denom

    return pl.pallas_call(
        body,
        grid=(B // K,),
        in_specs=[
            pl.BlockSpec((K, L), lambda i: (i, 0)),
            pl.BlockSpec((K, D), lambda i: (i, 0)),
            pl.BlockSpec((K, D), lambda i: (i, 0)),
        ],
        out_specs=pl.BlockSpec((K, 2 * D), lambda i: (i, 0)),
        out_shape=jax.ShapeDtypeStruct((B, 2 * D), jnp.float32),
    )(token_ids, title_emb, pooled)


def kernel(title_ids, token_ids, title_table, token_table):
    token_table_z = token_table.at[0].set(0.0)
    row2item = (jnp.arange(IPW * L, dtype=jnp.int32) // L).reshape(TOK_W, GW)
    title_emb, pooled = _sc_gather_pool(
        title_table,
        title_ids.astype(jnp.int32),
        token_table_z,
        token_ids.astype(jnp.int32).reshape(B * L),
        row2item,
    )
    return _tc_combine(token_ids, title_emb, pooled)


# speedup vs baseline: 9.1749x; 1.2114x over previous
import functools

import jax
import jax.numpy as jnp
from jax import lax
from jax.experimental import pallas as pl
from jax.experimental.pallas import tpu as pltpu
from jax.experimental.pallas import tpu_sc as plsc

B = 16384
V = 100001
T = 10000
D = 32
L = 20

NC = 2
NS = 16
NW = NC * NS
IPW = B // NW
GW = 128
TOK_W = IPW * L // GW
TIT_W = IPW // GW
NBUF = 4


def _sc_gather_pool(title_table, title_idx_3d, token_table_z, token_idx_3d,
                    row2item):
    mesh = plsc.VectorSubcoreMesh(core_axis_name="c", subcore_axis_name="s")
    out_type = (
        jax.ShapeDtypeStruct((B, D), jnp.float32),
        jax.ShapeDtypeStruct((B, D), jnp.float32),
    )

    @functools.partial(
        pl.kernel, out_type=out_type, mesh=mesh,
        scratch_types=[
            pltpu.VMEM((TOK_W, GW), jnp.int32),
            pltpu.VMEM((TOK_W, GW), jnp.int32),
            pltpu.VMEM((TIT_W, GW), jnp.int32),
            pltpu.VMEM((GW, D), jnp.float32),
            pltpu.VMEM((NBUF, GW, D), jnp.float32),
            pltpu.VMEM((TIT_W, GW, D), jnp.float32),
            pltpu.VMEM_SHARED((NS * IPW, D), jnp.float32),
            pltpu.SemaphoreType.DMA,
            pltpu.SemaphoreType.DMA,
            pltpu.SemaphoreType.DMA,
            pltpu.SemaphoreType.DMA,
            pltpu.SemaphoreType.DMA,
        ],
        compiler_params=pltpu.CompilerParams(use_tc_tiling_on_sc=False))
    def k(title_tab_hbm, title_idx_hbm, tok_tab_hbm, tok_idx_hbm, pat_hbm,
          out_title_hbm, out_pool_hbm,
          pat_v, idxa_v, tidx_v, zbuf_v, rows_v, trows_v, acc_sh,
          s0, s1, s2, s3, st):
        sems = (s0, s1, s2, s3)
        sid = lax.axis_index("s")
        wid = sid * NC + lax.axis_index("c")

        pltpu.sync_copy(tok_idx_hbm.at[wid], idxa_v)
        pltpu.sync_copy(title_idx_hbm.at[wid], tidx_v)
        pltpu.sync_copy(pat_hbm, pat_v)

        for b in range(NBUF):
            pltpu.async_copy(tok_tab_hbm.at[idxa_v.at[b]], rows_v.at[b],
                             sems[b])
        for b in range(TIT_W):
            pltpu.async_copy(title_tab_hbm.at[tidx_v.at[b]],
                             trows_v.at[b], st)

        off = jnp.broadcast_to(sid * IPW, (16,)).astype(jnp.int32)

        def obody(i, carry):
            a = i // (GW // 16)
            bb = i % (GW // 16)
            pat_v[a, pl.ds(bb * 16, 16)] = pat_v[a, pl.ds(bb * 16, 16)] + off
            return carry

        lax.fori_loop(0, TOK_W * (GW // 16), obody, 0)

        zero = jnp.zeros((16,), jnp.float32)

        def zbody(i, carry):
            zbuf_v[i, pl.ds(0, 16)] = zero
            zbuf_v[i, pl.ds(16, 16)] = zero
            return carry

        lax.fori_loop(0, GW, zbody, 0)

        def zcopy(c, carry):
            pltpu.sync_copy(zbuf_v, acc_sh.at[pl.ds(sid * IPW + c * GW, GW)])
            return carry

        lax.fori_loop(0, IPW // GW, zcopy, 0)

        for b in range(NBUF):
            pltpu.make_async_copy(
                tok_tab_hbm.at[idxa_v.at[b]], rows_v.at[b], sems[b]).wait()
            pltpu.sync_copy(rows_v.at[b], acc_sh.at[pat_v.at[b]], add=True)

        def gbody(w, carry):
            pltpu.async_copy(tok_tab_hbm.at[idxa_v.at[w]], rows_v.at[0],
                             sems[0]).wait()
            pltpu.sync_copy(rows_v.at[0], acc_sh.at[pat_v.at[w]], add=True)
            return carry

        lax.fori_loop(NBUF, TOK_W, gbody, 0)

        pltpu.sync_copy(acc_sh.at[pl.ds(sid * IPW, IPW)],
                        out_pool_hbm.at[pl.ds(wid * IPW, IPW)])

        for b in range(TIT_W):
            pltpu.make_async_copy(title_tab_hbm.at[tidx_v.at[b]],
                                  trows_v.at[b], st).wait()
        for b in range(TIT_W):
            pltpu.sync_copy(trows_v.at[b],
                            out_title_hbm.at[pl.ds(wid * IPW + b * GW, GW)])

    return k(title_table, title_idx_3d, token_table_z, token_idx_3d, row2item)


def _tc_combine(token_ids, title_emb, pooled):
    K = 1024

    def body(ids_ref, title_ref, pool_ref, out_ref):
        ids = ids_ref[...]
        cnt = jnp.sum((ids != 0).astype(jnp.float32), axis=1, keepdims=True)
        denom = jnp.maximum(cnt, 1.0)
        out_ref[:, :D] = title_ref[...]
        out_ref[:, D:] = pool_ref[...] / denom

    return pl.pallas_call(
        body,
        grid=(B // K,),
        in_specs=[
            pl.BlockSpec((K, L), lambda i: (i, 0)),
            pl.BlockSpec((K, D), lambda i: (i, 0)),
            pl.BlockSpec((K, D), lambda i: (i, 0)),
        ],
        out_specs=pl.BlockSpec((K, 2 * D), lambda i: (i, 0)),
        out_shape=jax.ShapeDtypeStruct((B, 2 * D), jnp.float32),
    )(token_ids, title_emb, pooled)


def kernel(title_ids, token_ids, title_table, token_table):
    token_table_z = token_table.at[0].set(0.0)
    row2item = (jnp.arange(IPW * L, dtype=jnp.int32) // L).reshape(TOK_W, GW)
    title_emb, pooled = _sc_gather_pool(
        title_table,
        title_ids.astype(jnp.int32).reshape(NW, TIT_W, GW),
        token_table_z,
        token_ids.astype(jnp.int32).reshape(NW, TOK_W, GW),
        row2item,
    )
    return _tc_combine(token_ids, title_emb, pooled)


# speedup vs baseline: 9.1952x; 1.0022x over previous
import functools

import jax
import jax.numpy as jnp
from jax import lax
from jax.experimental import pallas as pl
from jax.experimental.pallas import tpu as pltpu
from jax.experimental.pallas import tpu_sc as plsc

B = 16384
V = 100001
T = 10000
D = 32
L = 20

NC = 2
NS = 16
NW = NC * NS
IPW = B // NW
GW = 128
TOK_W = IPW * L // GW
TIT_W = IPW // GW
NBUF = 4


def _sc_gather_pool(title_table, title_idx_3d, token_table_z, token_idx_3d,
                    row2item):
    mesh = plsc.VectorSubcoreMesh(core_axis_name="c", subcore_axis_name="s")
    out_type = (
        jax.ShapeDtypeStruct((B, D), jnp.float32),
        jax.ShapeDtypeStruct((B, D), jnp.float32),
    )

    @functools.partial(
        pl.kernel, out_type=out_type, mesh=mesh,
        scratch_types=[
            pltpu.VMEM((TOK_W, GW), jnp.int32),
            pltpu.VMEM((TOK_W, GW), jnp.int32),
            pltpu.VMEM((TIT_W, GW), jnp.int32),
            pltpu.VMEM((GW, D), jnp.float32),
            pltpu.VMEM((NBUF, GW, D), jnp.float32),
            pltpu.VMEM((TIT_W, GW, D), jnp.float32),
            pltpu.VMEM_SHARED((NS * IPW + NS, D), jnp.float32),
            pltpu.SemaphoreType.DMA,
            pltpu.SemaphoreType.DMA,
            pltpu.SemaphoreType.DMA,
            pltpu.SemaphoreType.DMA,
            pltpu.SemaphoreType.DMA,
        ],
        compiler_params=pltpu.CompilerParams(use_tc_tiling_on_sc=False))
    def k(title_tab_hbm, title_idx_hbm, tok_tab_hbm, tok_idx_hbm, pat_hbm,
          out_title_hbm, out_pool_hbm,
          pat_v, idxa_v, tidx_v, zbuf_v, rows_v, trows_v, acc_sh,
          s0, s1, s2, s3, st):
        sems = (s0, s1, s2, s3)
        sid = lax.axis_index("s")
        wid = sid * NC + lax.axis_index("c")

        pltpu.sync_copy(tok_idx_hbm.at[wid], idxa_v)
        pltpu.sync_copy(title_idx_hbm.at[wid], tidx_v)
        pltpu.sync_copy(pat_hbm, pat_v)

        for b in range(NBUF):
            pltpu.async_copy(tok_tab_hbm.at[idxa_v.at[b]], rows_v.at[b],
                             sems[b])
        for b in range(TIT_W):
            pltpu.async_copy(title_tab_hbm.at[tidx_v.at[b]],
                             trows_v.at[b], st)

        off = jnp.broadcast_to(sid * IPW, (16,)).astype(jnp.int32)
        trash = jnp.broadcast_to(NS * IPW + sid, (16,)).astype(jnp.int32)

        def obody(i, carry):
            a = i // (GW // 16)
            bb = i % (GW // 16)
            ids = idxa_v[a, pl.ds(bb * 16, 16)]
            pat_v[a, pl.ds(bb * 16, 16)] = jnp.where(
                ids == 0, trash, pat_v[a, pl.ds(bb * 16, 16)] + off)
            return carry

        lax.fori_loop(0, TOK_W * (GW // 16), obody, 0)

        zero = jnp.zeros((16,), jnp.float32)

        def zbody(i, carry):
            zbuf_v[i, pl.ds(0, 16)] = zero
            zbuf_v[i, pl.ds(16, 16)] = zero
            return carry

        lax.fori_loop(0, GW, zbody, 0)

        def zcopy(c, carry):
            pltpu.sync_copy(zbuf_v, acc_sh.at[pl.ds(sid * IPW + c * GW, GW)])
            return carry

        lax.fori_loop(0, IPW // GW, zcopy, 0)

        for b in range(NBUF):
            pltpu.make_async_copy(
                tok_tab_hbm.at[idxa_v.at[b]], rows_v.at[b], sems[b]).wait()
            pltpu.sync_copy(rows_v.at[b], acc_sh.at[pat_v.at[b]], add=True)

        def gbody(w, carry):
            pltpu.async_copy(tok_tab_hbm.at[idxa_v.at[w]], rows_v.at[0],
                             sems[0]).wait()
            pltpu.sync_copy(rows_v.at[0], acc_sh.at[pat_v.at[w]], add=True)
            return carry

        lax.fori_loop(NBUF, TOK_W, gbody, 0)

        pltpu.sync_copy(acc_sh.at[pl.ds(sid * IPW, IPW)],
                        out_pool_hbm.at[pl.ds(wid * IPW, IPW)])

        for b in range(TIT_W):
            pltpu.make_async_copy(title_tab_hbm.at[tidx_v.at[b]],
                                  trows_v.at[b], st).wait()
        for b in range(TIT_W):
            pltpu.sync_copy(trows_v.at[b],
                            out_title_hbm.at[pl.ds(wid * IPW + b * GW, GW)])

    return k(title_table, title_idx_3d, token_table_z, token_idx_3d, row2item)


def _tc_combine(token_ids, title_emb, pooled):
    K = 1024

    def body(ids_ref, title_ref, pool_ref, out_ref):
        ids = ids_ref[...]
        cnt = jnp.sum((ids != 0).astype(jnp.float32), axis=1, keepdims=True)
        denom = jnp.maximum(cnt, 1.0)
        out_ref[:, :D] = title_ref[...]
        out_ref[:, D:] = pool_ref[...] / denom

    return pl.pallas_call(
        body,
        grid=(B // K,),
        in_specs=[
            pl.BlockSpec((K, L), lambda i: (i, 0)),
            pl.BlockSpec((K, D), lambda i: (i, 0)),
            pl.BlockSpec((K, D), lambda i: (i, 0)),
        ],
        out_specs=pl.BlockSpec((K, 2 * D), lambda i: (i, 0)),
        out_shape=jax.ShapeDtypeStruct((B, 2 * D), jnp.float32),
    )(token_ids, title_emb, pooled)


def kernel(title_ids, token_ids, title_table, token_table):
    row2item = (jnp.arange(IPW * L, dtype=jnp.int32) // L).reshape(TOK_W, GW)
    title_emb, pooled = _sc_gather_pool(
        title_table,
        title_ids.astype(jnp.int32).reshape(NW, TIT_W, GW),
        token_table,
        token_ids.astype(jnp.int32).reshape(NW, TOK_W, GW),
        row2item,
    )
    return _tc_combine(token_ids, title_emb, pooled)


# speedup vs baseline: 12.2646x; 1.3338x over previous
import functools

import jax
import jax.numpy as jnp
from jax import lax
from jax.experimental import pallas as pl
from jax.experimental.pallas import tpu as pltpu
from jax.experimental.pallas import tpu_sc as plsc

B = 16384
V = 100001
T = 10000
D = 32
L = 20

NC = 2
NS = 16
NW = NC * NS
IPW = B // NW
GW = 128
TOK_W = IPW * L // GW
TIT_W = IPW // GW
NBUF = 4


def _sc_gather_pool(title_table, title_idx_3d, token_table_z, token_idx_3d,
                    row2item):
    mesh = plsc.VectorSubcoreMesh(core_axis_name="c", subcore_axis_name="s")
    out_type = (
        jax.ShapeDtypeStruct((B, D), jnp.float32),
        jax.ShapeDtypeStruct((B, D), jnp.float32),
    )

    @functools.partial(
        pl.kernel, out_type=out_type, mesh=mesh,
        scratch_types=[
            pltpu.VMEM((TOK_W, GW), jnp.int32),
            pltpu.VMEM((TOK_W, GW), jnp.int32),
            pltpu.VMEM((TIT_W, GW), jnp.int32),
            pltpu.VMEM((GW, D), jnp.float32),
            pltpu.VMEM((NBUF, GW, D), jnp.float32),
            pltpu.VMEM((TIT_W, GW, D), jnp.float32),
            pltpu.VMEM_SHARED((NS * IPW + NS, D), jnp.float32),
            pltpu.SemaphoreType.DMA,
            pltpu.SemaphoreType.DMA,
            pltpu.SemaphoreType.DMA,
            pltpu.SemaphoreType.DMA,
            pltpu.SemaphoreType.DMA,
        ],
        compiler_params=pltpu.CompilerParams(use_tc_tiling_on_sc=False))
    def k(title_tab_hbm, title_idx_hbm, tok_tab_hbm, tok_idx_hbm, pat_hbm,
          out_title_hbm, out_pool_hbm,
          pat_v, idxa_v, tidx_v, zbuf_v, rows_v, trows_v, acc_sh,
          s0, s1, s2, s3, st):
        sems = (s0, s1, s2, s3)
        sid = lax.axis_index("s")
        wid = sid * NC + lax.axis_index("c")

        pltpu.sync_copy(tok_idx_hbm.at[wid], idxa_v)
        pltpu.sync_copy(title_idx_hbm.at[wid], tidx_v)
        pltpu.sync_copy(pat_hbm, pat_v)

        for b in range(NBUF):
            pltpu.async_copy(tok_tab_hbm.at[idxa_v.at[b]], rows_v.at[b],
                             sems[b])
        for b in range(TIT_W):
            pltpu.async_copy(title_tab_hbm.at[tidx_v.at[b]],
                             trows_v.at[b], st)

        off = jnp.broadcast_to(sid * IPW, (16,)).astype(jnp.int32)
        trash = jnp.broadcast_to(NS * IPW + sid, (16,)).astype(jnp.int32)

        def obody(i, carry):
            a = i // (GW // 16)
            bb = i % (GW // 16)
            ids = idxa_v[a, pl.ds(bb * 16, 16)]
            pat_v[a, pl.ds(bb * 16, 16)] = jnp.where(
                ids == 0, trash, pat_v[a, pl.ds(bb * 16, 16)] + off)
            return carry

        lax.fori_loop(0, TOK_W * (GW // 16), obody, 0)

        zero = jnp.zeros((16,), jnp.float32)

        def zbody(i, carry):
            zbuf_v[i, pl.ds(0, 16)] = zero
            zbuf_v[i, pl.ds(16, 16)] = zero
            return carry

        lax.fori_loop(0, GW, zbody, 0)

        def zcopy(c, carry):
            pltpu.sync_copy(zbuf_v, acc_sh.at[pl.ds(sid * IPW + c * GW, GW)])
            return carry

        lax.fori_loop(0, IPW // GW, zcopy, 0)

        def gbody(g, carry):
            for b in range(NBUF):
                w = g * NBUF + b
                pltpu.make_async_copy(
                    tok_tab_hbm.at[idxa_v.at[w]], rows_v.at[b], sems[b]).wait()
                pltpu.sync_copy(rows_v.at[b], acc_sh.at[pat_v.at[w]], add=True)
                pltpu.async_copy(tok_tab_hbm.at[idxa_v.at[w + NBUF]],
                                 rows_v.at[b], sems[b])
            return carry

        lax.fori_loop(0, TOK_W // NBUF - 1, gbody, 0)

        for b in range(NBUF):
            w = TOK_W - NBUF + b
            pltpu.make_async_copy(
                tok_tab_hbm.at[idxa_v.at[w]], rows_v.at[b], sems[b]).wait()
            pltpu.sync_copy(rows_v.at[b], acc_sh.at[pat_v.at[w]], add=True)

        pltpu.sync_copy(acc_sh.at[pl.ds(sid * IPW, IPW)],
                        out_pool_hbm.at[pl.ds(wid * IPW, IPW)])

        for b in range(TIT_W):
            pltpu.make_async_copy(title_tab_hbm.at[tidx_v.at[b]],
                                  trows_v.at[b], st).wait()
        for b in range(TIT_W):
            pltpu.sync_copy(trows_v.at[b],
                            out_title_hbm.at[pl.ds(wid * IPW + b * GW, GW)])

    return k(title_table, title_idx_3d, token_table_z, token_idx_3d, row2item)


def _tc_combine(token_ids, title_emb, pooled):
    K = 1024

    def body(ids_ref, title_ref, pool_ref, out_ref):
        ids = ids_ref[...]
        cnt = jnp.sum((ids != 0).astype(jnp.float32), axis=1, keepdims=True)
        denom = jnp.maximum(cnt, 1.0)
        out_ref[:, :D] = title_ref[...]
        out_ref[:, D:] = pool_ref[...] / denom

    return pl.pallas_call(
        body,
        grid=(B // K,),
        in_specs=[
            pl.BlockSpec((K, L), lambda i: (i, 0)),
            pl.BlockSpec((K, D), lambda i: (i, 0)),
            pl.BlockSpec((K, D), lambda i: (i, 0)),
        ],
        out_specs=pl.BlockSpec((K, 2 * D), lambda i: (i, 0)),
        out_shape=jax.ShapeDtypeStruct((B, 2 * D), jnp.float32),
    )(token_ids, title_emb, pooled)


def kernel(title_ids, token_ids, title_table, token_table):
    row2item = (jnp.arange(IPW * L, dtype=jnp.int32) // L).reshape(TOK_W, GW)
    title_emb, pooled = _sc_gather_pool(
        title_table,
        title_ids.astype(jnp.int32).reshape(NW, TIT_W, GW),
        token_table,
        token_ids.astype(jnp.int32).reshape(NW, TOK_W, GW),
        row2item,
    )
    return _tc_combine(token_ids, title_emb, pooled)


# speedup vs baseline: 12.2666x; 1.0002x over previous
import functools

import jax
import jax.numpy as jnp
from jax import lax
from jax.experimental import pallas as pl
from jax.experimental.pallas import tpu as pltpu
from jax.experimental.pallas import tpu_sc as plsc

B = 16384
V = 100001
T = 10000
D = 32
L = 20

NC = 2
NS = 16
NW = NC * NS
IPW = B // NW
TGW = 80
TOK_W = IPW * L // TGW
TITGW = 128
TIT_W = IPW // TITGW
NB = 8


def _sc_gather_pool(title_table, title_idx_3d, token_table, token_idx_3d,
                    row2item):
    mesh = plsc.VectorSubcoreMesh(core_axis_name="c", subcore_axis_name="s")
    out_type = (
        jax.ShapeDtypeStruct((B, D), jnp.float32),
        jax.ShapeDtypeStruct((B, D), jnp.float32),
    )

    @functools.partial(
        pl.kernel, out_type=out_type, mesh=mesh,
        scratch_types=[
            pltpu.VMEM((TOK_W, TGW), jnp.int32),
            pltpu.VMEM((TOK_W, TGW), jnp.int32),
            pltpu.VMEM((TIT_W, TITGW), jnp.int32),
            pltpu.VMEM((TITGW, D), jnp.float32),
            pltpu.VMEM((NB, TGW, D), jnp.float32),
            pltpu.VMEM((TIT_W, TITGW, D), jnp.float32),
            pltpu.VMEM_SHARED((NS * IPW + NS, D), jnp.float32),
            pltpu.SemaphoreType.DMA,
            pltpu.SemaphoreType.DMA,
            pltpu.SemaphoreType.DMA,
            pltpu.SemaphoreType.DMA,
            pltpu.SemaphoreType.DMA,
            pltpu.SemaphoreType.DMA,
            pltpu.SemaphoreType.DMA,
            pltpu.SemaphoreType.DMA,
            pltpu.SemaphoreType.DMA,
            pltpu.SemaphoreType.DMA,
            pltpu.SemaphoreType.DMA,
            pltpu.SemaphoreType.DMA,
            pltpu.SemaphoreType.DMA,
            pltpu.SemaphoreType.DMA,
            pltpu.SemaphoreType.DMA,
            pltpu.SemaphoreType.DMA,
            pltpu.SemaphoreType.DMA,
        ],
        compiler_params=pltpu.CompilerParams(use_tc_tiling_on_sc=False))
    def k(title_tab_hbm, title_idx_hbm, tok_tab_hbm, tok_idx_hbm, pat_hbm,
          out_title_hbm, out_pool_hbm,
          pat_v, idxa_v, tidx_v, zbuf_v, rows_v, trows_v, acc_sh,
          g0, g1, g2, g3, g4, g5, g6, g7,
          c0, c1, c2, c3, c4, c5, c6, c7, st):
        gsem = (g0, g1, g2, g3, g4, g5, g6, g7)
        ssem = (c0, c1, c2, c3, c4, c5, c6, c7)
        sid = lax.axis_index("s")
        wid = sid * NC + lax.axis_index("c")

        pltpu.sync_copy(tok_idx_hbm.at[wid], idxa_v)
        pltpu.sync_copy(title_idx_hbm.at[wid], tidx_v)
        pltpu.sync_copy(pat_hbm, pat_v)

        for b in range(NB):
            pltpu.async_copy(tok_tab_hbm.at[idxa_v.at[b]], rows_v.at[b],
                             gsem[b])
        for b in range(TIT_W):
            pltpu.async_copy(title_tab_hbm.at[tidx_v.at[b]],
                             trows_v.at[b], st)

        off = jnp.broadcast_to(sid * IPW, (16,)).astype(jnp.int32)
        trash = jnp.broadcast_to(NS * IPW + sid, (16,)).astype(jnp.int32)

        def obody(i, carry):
            a = i // (TGW // 16)
            bb = i % (TGW // 16)
            ids = idxa_v[a, pl.ds(bb * 16, 16)]
            pat_v[a, pl.ds(bb * 16, 16)] = jnp.where(
                ids == 0, trash, pat_v[a, pl.ds(bb * 16, 16)] + off)
            return carry

        lax.fori_loop(0, TOK_W * (TGW // 16), obody, 0)

        zero = jnp.zeros((16,), jnp.float32)

        def zbody(i, carry):
            zbuf_v[i, pl.ds(0, 16)] = zero
            zbuf_v[i, pl.ds(16, 16)] = zero
            return carry

        lax.fori_loop(0, TITGW, zbody, 0)

        def zcopy(c, carry):
            pltpu.sync_copy(zbuf_v,
                            acc_sh.at[pl.ds(sid * IPW + c * TITGW, TITGW)])
            return carry

        lax.fori_loop(0, IPW // TITGW, zcopy, 0)

        def gbody(g, carry):
            for b in range(NB):
                w = g * NB + b
                pltpu.make_async_copy(
                    tok_tab_hbm.at[idxa_v.at[w]], rows_v.at[b],
                    gsem[b]).wait()
                pltpu.async_copy(rows_v.at[b], acc_sh.at[pat_v.at[w]],
                                 ssem[b], add=True)
            for b in range(NB):
                w = g * NB + b
                pltpu.make_async_copy(rows_v.at[b], acc_sh.at[pat_v.at[w]],
                                      ssem[b]).wait()
                pltpu.async_copy(tok_tab_hbm.at[idxa_v.at[w + NB]],
                                 rows_v.at[b], gsem[b])
            return carry

        lax.fori_loop(0, TOK_W // NB - 1, gbody, 0)

        for b in range(NB):
            w = TOK_W - NB + b
            pltpu.make_async_copy(
                tok_tab_hbm.at[idxa_v.at[w]], rows_v.at[b], gsem[b]).wait()
            pltpu.async_copy(rows_v.at[b], acc_sh.at[pat_v.at[w]],
                             ssem[b], add=True)
        for b in range(NB):
            w = TOK_W - NB + b
            pltpu.make_async_copy(rows_v.at[b], acc_sh.at[pat_v.at[w]],
                                  ssem[b]).wait()

        pltpu.sync_copy(acc_sh.at[pl.ds(sid * IPW, IPW)],
                        out_pool_hbm.at[pl.ds(wid * IPW, IPW)])

        for b in range(TIT_W):
            pltpu.make_async_copy(title_tab_hbm.at[tidx_v.at[b]],
                                  trows_v.at[b], st).wait()
        for b in range(TIT_W):
            pltpu.sync_copy(trows_v.at[b],
                            out_title_hbm.at[pl.ds(wid * IPW + b * TITGW,
                                                   TITGW)])

    return k(title_table, title_idx_3d, token_table, token_idx_3d, row2item)


def _tc_combine(token_ids, title_emb, pooled):
    K = 1024

    def body(ids_ref, title_ref, pool_ref, out_ref):
        ids = ids_ref[...]
        cnt = jnp.sum((ids != 0).astype(jnp.float32), axis=1, keepdims=True)
        denom = jnp.maximum(cnt, 1.0)
        out_ref[:, :D] = title_ref[...]
        out_ref[:, D:] = pool_ref[...] / denom

    return pl.pallas_call(
        body,
        grid=(B // K,),
        in_specs=[
            pl.BlockSpec((K, L), lambda i: (i, 0)),
            pl.BlockSpec((K, D), lambda i: (i, 0)),
            pl.BlockSpec((K, D), lambda i: (i, 0)),
        ],
        out_specs=pl.BlockSpec((K, 2 * D), lambda i: (i, 0)),
        out_shape=jax.ShapeDtypeStruct((B, 2 * D), jnp.float32),
    )(token_ids, title_emb, pooled)


def kernel(title_ids, token_ids, title_table, token_table):
    row2item = (jnp.arange(IPW * L, dtype=jnp.int32) // L).reshape(TOK_W, TGW)
    title_emb, pooled = _sc_gather_pool(
        title_table,
        title_ids.astype(jnp.int32).reshape(NW, TIT_W, TITGW),
        token_table,
        token_ids.astype(jnp.int32).reshape(NW, TOK_W, TGW),
        row2item,
    )
    return _tc_combine(token_ids, title_emb, pooled)


# speedup vs baseline: 12.2981x; 1.0026x over previous
import functools

import jax
import jax.numpy as jnp
from jax import lax
from jax.experimental import pallas as pl
from jax.experimental.pallas import tpu as pltpu
from jax.experimental.pallas import tpu_sc as plsc

B = 16384
V = 100001
T = 10000
D = 32
L = 20

NC = 2
NS = 16
NW = NC * NS
IPW = B // NW
GW = 128
TOK_W = IPW * L // GW
TIT_W = IPW // GW
NBUF = 4


def _sc_gather_pool(title_table, title_idx_3d, token_table_z, token_idx_3d,
                    row2item):
    mesh = plsc.VectorSubcoreMesh(core_axis_name="c", subcore_axis_name="s")
    out_type = (
        jax.ShapeDtypeStruct((B, D), jnp.float32),
        jax.ShapeDtypeStruct((B, D), jnp.float32),
    )

    @functools.partial(
        pl.kernel, out_type=out_type, mesh=mesh,
        scratch_types=[
            pltpu.VMEM((TOK_W, GW), jnp.int32),
            pltpu.VMEM((TOK_W, GW), jnp.int32),
            pltpu.VMEM((TIT_W, GW), jnp.int32),
            pltpu.VMEM((GW, D), jnp.float32),
            pltpu.VMEM((NBUF, GW, D), jnp.float32),
            pltpu.VMEM((TIT_W, GW, D), jnp.float32),
            pltpu.VMEM_SHARED((NS * IPW + NS, D), jnp.float32),
            pltpu.SemaphoreType.DMA,
            pltpu.SemaphoreType.DMA,
            pltpu.SemaphoreType.DMA,
            pltpu.SemaphoreType.DMA,
            pltpu.SemaphoreType.DMA,
        ],
        compiler_params=pltpu.CompilerParams(use_tc_tiling_on_sc=False))
    def k(title_tab_hbm, title_idx_hbm, tok_tab_hbm, tok_idx_hbm, pat_hbm,
          out_title_hbm, out_pool_hbm,
          pat_v, idxa_v, tidx_v, zbuf_v, rows_v, trows_v, acc_sh,
          s0, s1, s2, s3, st):
        sems = (s0, s1, s2, s3)
        sid = lax.axis_index("s")
        wid = sid * NC + lax.axis_index("c")

        pltpu.sync_copy(tok_idx_hbm.at[wid], idxa_v)
        pltpu.sync_copy(title_idx_hbm.at[wid], tidx_v)
        pltpu.sync_copy(pat_hbm, pat_v)

        for b in range(NBUF):
            pltpu.async_copy(tok_tab_hbm.at[idxa_v.at[b]], rows_v.at[b],
                             sems[b])
        for b in range(TIT_W):
            pltpu.async_copy(title_tab_hbm.at[tidx_v.at[b]],
                             trows_v.at[b], st)

        off = jnp.broadcast_to(sid * IPW, (16,)).astype(jnp.int32)
        trash = jnp.broadcast_to(NS * IPW + sid, (16,)).astype(jnp.int32)

        def obody(i, carry):
            a = i // (GW // 16)
            bb = i % (GW // 16)
            ids = idxa_v[a, pl.ds(bb * 16, 16)]
            pat_v[a, pl.ds(bb * 16, 16)] = jnp.where(
                ids == 0, trash, pat_v[a, pl.ds(bb * 16, 16)] + off)
            return carry

        lax.fori_loop(0, TOK_W * (GW // 16), obody, 0)

        zero = jnp.zeros((16,), jnp.float32)

        def zbody(i, carry):
            zbuf_v[i, pl.ds(0, 16)] = zero
            zbuf_v[i, pl.ds(16, 16)] = zero
            return carry

        lax.fori_loop(0, GW, zbody, 0)

        def zcopy(c, carry):
            pltpu.sync_copy(zbuf_v, acc_sh.at[pl.ds(sid * IPW + c * GW, GW)])
            return carry

        lax.fori_loop(0, IPW // GW, zcopy, 0)

        def gbody(g, carry):
            for b in range(NBUF):
                w = g * NBUF + b
                pltpu.make_async_copy(
                    tok_tab_hbm.at[idxa_v.at[w]], rows_v.at[b], sems[b]).wait()
                pltpu.sync_copy(rows_v.at[b], acc_sh.at[pat_v.at[w]], add=True)
                pltpu.async_copy(tok_tab_hbm.at[idxa_v.at[w + NBUF]],
                                 rows_v.at[b], sems[b])
            return carry

        lax.fori_loop(0, TOK_W // NBUF - 1, gbody, 0)

        for b in range(NBUF):
            w = TOK_W - NBUF + b
            pltpu.make_async_copy(
                tok_tab_hbm.at[idxa_v.at[w]], rows_v.at[b], sems[b]).wait()
            pltpu.sync_copy(rows_v.at[b], acc_sh.at[pat_v.at[w]], add=True)

        pltpu.sync_copy(acc_sh.at[pl.ds(sid * IPW, IPW)],
                        out_pool_hbm.at[pl.ds(wid * IPW, IPW)])

        for b in range(TIT_W):
            pltpu.make_async_copy(title_tab_hbm.at[tidx_v.at[b]],
                                  trows_v.at[b], st).wait()
        for b in range(TIT_W):
            pltpu.sync_copy(trows_v.at[b],
                            out_title_hbm.at[pl.ds(wid * IPW + b * GW, GW)])

    return k(title_table, title_idx_3d, token_table_z, token_idx_3d, row2item)


def _tc_combine(token_ids, title_emb, pooled):
    K = 1024

    def body(ids_ref, title_ref, pool_ref, out_ref):
        ids = ids_ref[...]
        cnt = jnp.sum((ids != 0).astype(jnp.float32), axis=1, keepdims=True)
        denom = jnp.maximum(cnt, 1.0)
        out_ref[:, :D] = title_ref[...]
        out_ref[:, D:] = pool_ref[...] / denom

    return pl.pallas_call(
        body,
        grid=(B // K,),
        in_specs=[
            pl.BlockSpec((K, L), lambda i: (i, 0)),
            pl.BlockSpec((K, D), lambda i: (i, 0)),
            pl.BlockSpec((K, D), lambda i: (i, 0)),
        ],
        out_specs=pl.BlockSpec((K, 2 * D), lambda i: (i, 0)),
        out_shape=jax.ShapeDtypeStruct((B, 2 * D), jnp.float32),
    )(token_ids, title_emb, pooled)


def kernel(title_ids, token_ids, title_table, token_table):
    row2item = (jnp.arange(IPW * L, dtype=jnp.int32) // L).reshape(TOK_W, GW)
    title_emb, pooled = _sc_gather_pool(
        title_table,
        title_ids.astype(jnp.int32).reshape(NW, TIT_W, GW),
        token_table,
        token_ids.astype(jnp.int32).reshape(NW, TOK_W, GW),
        row2item,
    )
    return _tc_combine(token_ids, title_emb, pooled)
